# baseline (speedup 1.0000x reference)
import functools

import jax
import jax.numpy as jnp
import numpy as np
from jax import lax
from jax.experimental import pallas as pl
from jax.experimental.pallas import tpu as pltpu
from jax.experimental.pallas import tpu_sc as plsc

N_SAMPLES = 1000
REACH = 10.0
MAX_DIST = 0.05
IMG = 512

NC = 2
NS = 16
NW = NC * NS
PT = 128
PPAD = NW * PT
NCHUNK = 64
SPAD = NCHUNK * 16
BIGF = np.float32(3.4e38)
BIGI = np.int32(2**30)


def _vsqrt(x):
    r = lax.bitcast_convert_type(
        jnp.int32(0x5F3759DF) - (lax.bitcast_convert_type(x, jnp.int32) >> 1),
        jnp.float32)
    r = r * (1.5 - 0.5 * x * r * r)
    r = r * (1.5 - 0.5 * x * r * r)
    r = r * (1.5 - 0.5 * x * r * r)
    s = x * r
    return 0.5 * (s + x / s)


def _vsqrt_precise(x):
    s = _vsqrt(x)
    c = jnp.float32(4097.0)
    hi = (s * c) - ((s * c) - s)
    lo = s - hi
    s2 = hi * hi + (2.0 * hi * lo + lo * lo)
    resid = (x - s2)
    return s + resid / (2.0 * s)


_DNUMS = lax.GatherDimensionNumbers(
    offset_dims=(), collapsed_slice_dims=(0,), start_index_map=(0,))


def _perm(x, p):
    return lax.gather(x, p[:, None], _DNUMS, (1,),
                      indices_are_sorted=False, unique_indices=False,
                      mode=lax.GatherScatterMode.PROMISE_IN_BOUNDS)


def _lane_argmin(val, idx, i16):
    for s in (8, 4, 2, 1):
        p = i16 ^ s
        sv = _perm(val, p)
        si = _perm(idx, p)
        take = (sv < val) | ((sv == val) & (si < idx))
        val = jnp.where(take, sv, val)
        idx = jnp.where(take, si, idx)
    return val[0], idx[0]


def _sc_kernel_body(n_valid, quad_hbm, segs_hbm, t_hbm, si_hbm, lin_hbm,
                    outdx_hbm, outdy_hbm, outsum_hbm,
                    lin_v, segs_v, tt_v, si_v,
                    idx0_v, idx1_v, idx2_v, idx3_v,
                    land0_v, land1_v, land2_v, land3_v,
                    wx0_v, wx1_v, wx2_v, wx3_v,
                    wy0_v, wy1_v, wy2_v, wy3_v,
                    dd0_v, dd1_v, dd2_v, dd3_v,
                    outx_v, outy_v, sums_v, sem0, sem1, sem2, sem3):
    wid = lax.axis_index("s") * NC + lax.axis_index("c")
    base = wid * PT
    i16 = lax.iota(jnp.int32, 16)
    rep4 = i16 >> 2
    lane4 = i16 & 3
    m1 = (i16 & 1) == 1
    m2 = (i16 & 2) == 2

    pltpu.sync_copy(lin_hbm, lin_v.at[pl.ds(0, SPAD)])
    pltpu.sync_copy(segs_hbm, segs_v.at[pl.ds(0, 1024)])
    pltpu.sync_copy(t_hbm.at[pl.ds(base, PT)], tt_v.at[pl.ds(0, PT)])
    pltpu.sync_copy(si_hbm.at[pl.ds(base, PT)], si_v.at[pl.ds(0, PT)])

    for g in range(PT // 16):
        outx_v[pl.ds(g * 16, 16)] = jnp.zeros((16,), jnp.float32)
        outy_v[pl.ds(g * 16, 16)] = jnp.zeros((16,), jnp.float32)

    def prep(p):
        si = si_v[pl.ds(p, 16)][0]
        s4 = segs_v[pl.ds(si * 4, 16)]
        tt = jnp.full((16,), tt_v[pl.ds(p, 16)][0])
        rx0 = jnp.full((16,), s4[0])
        ry0 = jnp.full((16,), s4[1])
        rx1 = jnp.full((16,), s4[2])
        ry1 = jnp.full((16,), s4[3])
        ax0 = (rx0 * 0.5 + 0.5) * np.float32(IMG)
        ay0 = (0.5 - ry0 * 0.5) * np.float32(IMG)
        ax1 = (rx1 * 0.5 + 0.5) * np.float32(IMG)
        ay1 = (0.5 - ry1 * 0.5) * np.float32(IMG)
        px = (1.0 - tt) * ax0 + tt * ax1
        py = (1.0 - tt) * ay0 + tt * ay1
        dirx = ax1 - ax0
        diry = ay1 - ay0
        nx = 0.0 - diry
        ny = dirx
        nrm = _vsqrt_precise(jnp.maximum(nx * nx + ny * ny,
                                         jnp.float32(1e-37)))
        den = jnp.maximum(nrm, jnp.float32(1e-12))
        dvx = nx / den
        dvy = ny / den
        cx = jnp.abs(dvx) <= 1e-3
        cy = jnp.abs(dvy) <= 1e-3
        sdx = jnp.where(cx, jnp.float32(1.0), dvx)
        sdy = jnp.where(cy, jnp.float32(1.0), dvy)
        tl = jnp.where(cx, -BIGF, (0.0 - px) / sdx)
        tr = jnp.where(cx, BIGF, (np.float32(IMG - 1) - px) / sdx)
        tp = jnp.where(cy, -BIGF, (0.0 - py) / sdy)
        tb = jnp.where(cy, BIGF, (np.float32(IMG - 1) - py) / sdy)
        tmin = jnp.maximum(tl, tp)
        tmax = jnp.minimum(tr, tb)
        i1x = px + tmin * dvx
        i1y = py + tmin * dvy
        d12x = (px + tmax * dvx) - i1x
        d12y = (py + tmax * dvy) - i1y
        return i1x, i1y, d12x, d12y, px, py

    def a_side(p, idxS, wxS, wyS, ddS, landS, semS):
        i1x, i1y, d12x, d12y, px, py = prep(p)

        def phase_a(c, carry):
            dmin, dmini = carry
            sl = pl.ds(c * 16, 16)
            l = lin_v[sl]
            sx = l * d12x + i1x
            sy = l * d12y + i1y
            sx = jnp.minimum(jnp.maximum(sx, jnp.float32(0.0)),
                             np.float32(IMG - 1))
            sy = jnp.minimum(jnp.maximum(sy, jnp.float32(0.0)),
                             np.float32(IMG - 1))
            xi = sx.astype(jnp.int32)
            yi = sy.astype(jnp.int32)
            wxS[sl] = sx - xi.astype(jnp.float32)
            wyS[sl] = sy - yi.astype(jnp.float32)
            idxS[sl] = yi * IMG + xi
            ddx = sx - px
            ddy = sy - py
            dist = _vsqrt(jnp.maximum(ddx * ddx + ddy * ddy,
                                      jnp.float32(1e-37)))
            dist = jnp.where((ddx == 0.0) & (ddy == 0.0),
                             jnp.float32(0.0), dist)
            ddS[sl] = dist
            gi = c * 16 + i16
            dm = jnp.where(gi < N_SAMPLES, dist, BIGF)
            lt = dm < dmin
            dmin = jnp.where(lt, dm, dmin)
            dmini = jnp.where(lt, gi, dmini)
            return dmin, dmini

        dmin, dmini = lax.fori_loop(
            0, NCHUNK, phase_a,
            (jnp.full((16,), BIGF), jnp.full((16,), BIGI)), unroll=2)
        _, ref_ind = _lane_argmin(dmin, dmini, i16)
        pltpu.async_copy(quad_hbm.at[idxS], landS, semS)
        return ref_ind

    def b_side(p, ref_ind, idxS, wxS, wyS, ddS, landS, semS, sum_dd):
        pltpu.make_async_copy(quad_hbm.at[idxS], landS, semS).wait()

        def phase_b(c, carry):
            amin, amini = carry
            sl = pl.ds(c * 16, 16)
            wx = wxS[sl]
            wy = wyS[sl]
            dist = ddS[sl]
            vals16 = jnp.zeros((16,), jnp.float32)
            for b in range(4):
                s0 = c * 16 + 4 * b
                q0 = landS[s0, :]
                q1 = landS[s0 + 1, :]
                q2 = landS[s0 + 2, :]
                q3 = landS[s0 + 3, :]
                packed = jnp.where(
                    rep4 == 0, _perm(q0, lane4),
                    jnp.where(rep4 == 1, _perm(q1, lane4),
                              jnp.where(rep4 == 2, _perm(q2, lane4),
                                        _perm(q3, lane4))))
                wsel = 4 * b + rep4
                wxq = _perm(wx, wsel)
                wyq = _perm(wy, wsel)
                a = jnp.where(m1, wxq, 1.0 - wxq)
                bb = jnp.where(m2, wyq, 1.0 - wyq)
                prod = (a * bb) * packed
                t1 = prod + _perm(prod, i16 ^ 1)
                t2 = t1 + _perm(t1, i16 ^ 2)
                vals16 = jnp.where(rep4 == b,
                                   _perm(t2, 4 * ((i16 - 4 * b) & 3)),
                                   vals16)
            vals = 0.0 - vals16
            gi = c * 16 + i16
            vals = vals + jnp.where(gi == ref_ind, jnp.float32(-1e-6),
                                    jnp.float32(0.0))
            adj = vals * jnp.exp(-dist / np.float32(REACH))
            am = jnp.where(gi < N_SAMPLES, adj, BIGF)
            lt = am < amin
            amin = jnp.where(lt, am, amin)
            amini = jnp.where(lt, gi, amini)
            return amin, amini

        amin, amini = lax.fori_loop(
            0, NCHUNK, phase_b,
            (jnp.full((16,), BIGF), jnp.full((16,), BIGI)), unroll=2)
        _, m = _lane_argmin(amin, amini, i16)

        i1x, i1y, d12x, d12y, px, py = prep(p)
        lm = jnp.full((16,), lin_v[pl.ds(m, 16)][0])
        smx = jnp.minimum(jnp.maximum(lm * d12x + i1x, jnp.float32(0.0)),
                          np.float32(IMG - 1))
        smy = jnp.minimum(jnp.maximum(lm * d12y + i1y, jnp.float32(0.0)),
                          np.float32(IMG - 1))
        dex = smx - px
        dey = smy - py
        rx = dex / np.float32(IMG) * 2.0
        ry = dey / np.float32(IMG) * 2.0
        dd = _vsqrt(jnp.maximum(rx * rx + ry * ry, jnp.float32(1e-37)))
        dd = jnp.where((dex == 0.0) & (dey == 0.0), jnp.float32(0.0), dd)
        keep = dd < np.float32(MAX_DIST)
        gpid_v = jnp.full((16,), p * NW + wid, jnp.int32)
        keepv = keep & (gpid_v < n_valid)
        dex_m = jnp.where(keepv, dex, jnp.float32(0.0))
        dey_m = jnp.where(keepv, dey, jnp.float32(0.0))
        dd_m = jnp.where(keepv, dd, jnp.float32(0.0))
        row = (p >> 4) * 16
        lane = jnp.full((16,), p & 15, jnp.int32)
        ox = outx_v[pl.ds(row, 16)]
        oy = outy_v[pl.ds(row, 16)]
        outx_v[pl.ds(row, 16)] = jnp.where(i16 == lane, dex_m, ox)
        outy_v[pl.ds(row, 16)] = jnp.where(i16 == lane, dey_m, oy)
        return sum_dd + dd_m

    slots = ((idx0_v, wx0_v, wy0_v, dd0_v, land0_v, sem0),
             (idx1_v, wx1_v, wy1_v, dd1_v, land1_v, sem1),
             (idx2_v, wx2_v, wy2_v, dd2_v, land2_v, sem2),
             (idx3_v, wx3_v, wy3_v, dd3_v, land3_v, sem3))

    ri0 = a_side(jnp.int32(0), *slots[0])
    ri1 = a_side(jnp.int32(1), *slots[1])

    def pipe_body(i, carry):
        sum_dd, ri0, ri1 = carry
        p = i * 4
        ri2 = a_side(p + 2, *slots[2])
        ri3 = a_side(p + 3, *slots[3])
        sum_dd = b_side(p, ri0, *slots[0], sum_dd)
        sum_dd = b_side(p + 1, ri1, *slots[1], sum_dd)
        ri0 = a_side(p + 4, *slots[0])
        ri1 = a_side(p + 5, *slots[1])
        sum_dd = b_side(p + 2, ri2, *slots[2], sum_dd)
        sum_dd = b_side(p + 3, ri3, *slots[3], sum_dd)
        return sum_dd, ri0, ri1

    sum_dd, ri0, ri1 = lax.fori_loop(
        0, (PT - 8) // 4, pipe_body,
        (jnp.zeros((16,), jnp.float32), ri0, ri1), unroll=False)
    e = jnp.int32(PT - 8)
    ri2 = a_side(e + 2, *slots[2])
    ri3 = a_side(e + 3, *slots[3])
    sum_dd = b_side(e, ri0, *slots[0], sum_dd)
    sum_dd = b_side(e + 1, ri1, *slots[1], sum_dd)
    ri0 = a_side(e + 4, *slots[0])
    ri1 = a_side(e + 5, *slots[1])
    sum_dd = b_side(e + 2, ri2, *slots[2], sum_dd)
    sum_dd = b_side(e + 3, ri3, *slots[3], sum_dd)
    ri2 = a_side(e + 6, *slots[2])
    ri3 = a_side(e + 7, *slots[3])
    sum_dd = b_side(e + 4, ri0, *slots[0], sum_dd)
    sum_dd = b_side(e + 5, ri1, *slots[1], sum_dd)
    sum_dd = b_side(e + 6, ri2, *slots[2], sum_dd)
    sum_dd = b_side(e + 7, ri3, *slots[3], sum_dd)

    sums_v[pl.ds(0, 16)] = jnp.where(i16 == 0, sum_dd,
                                     jnp.zeros((16,), jnp.float32))
    pltpu.sync_copy(outx_v, outdx_hbm.at[pl.ds(base, PT)])
    pltpu.sync_copy(outy_v, outdy_hbm.at[pl.ds(base, PT)])
    pltpu.sync_copy(sums_v, outsum_hbm.at[pl.ds(wid * 16, 16)])


@functools.partial(jax.jit, static_argnums=(5,))
def _sc_call(quad, segs_flat, t_tiled, si_tiled, lin_pad, n_valid):
    mesh = plsc.VectorSubcoreMesh(core_axis_name="c", subcore_axis_name="s")
    f32 = jnp.float32
    kern = functools.partial(
        pl.kernel, mesh=mesh,
        compiler_params=pltpu.CompilerParams(use_tc_tiling_on_sc=False),
        out_type=[
            jax.ShapeDtypeStruct((PPAD,), f32),
            jax.ShapeDtypeStruct((PPAD,), f32),
            jax.ShapeDtypeStruct((NW * 16,), f32),
        ],
        scratch_types=(
            [pltpu.VMEM((SPAD + 16,), f32),
             pltpu.VMEM((1024 + 16,), f32),
             pltpu.VMEM((PT + 16,), f32),
             pltpu.VMEM((PT + 16,), jnp.int32)]
            + [pltpu.VMEM((SPAD,), jnp.int32)] * 4
            + [pltpu.VMEM((SPAD, 16), f32)] * 4
            + [pltpu.VMEM((SPAD,), f32)] * 12
            + [pltpu.VMEM((PT,), f32),
               pltpu.VMEM((PT,), f32),
               pltpu.VMEM((16,), f32)]
            + [pltpu.SemaphoreType.DMA] * 4
        ),
    )(functools.partial(_sc_kernel_body, jnp.int32(n_valid)))
    return kern(quad, segs_flat, t_tiled, si_tiled, lin_pad)


def kernel(edge_segments, edge_image, t, seg_idx):
    P = t.shape[0]
    imx = jnp.concatenate([edge_image[:, 1:], edge_image[:, -1:]], axis=1)
    imy = jnp.concatenate([edge_image[1:, :], edge_image[-1:, :]], axis=0)
    imxy = jnp.concatenate([imy[:, 1:], imy[:, -1:]], axis=1)
    quad = jnp.stack([edge_image.reshape(-1), imx.reshape(-1),
                      imy.reshape(-1), imxy.reshape(-1)], axis=1)
    quad = jnp.pad(quad, ((0, 0), (0, 12)))
    segs_flat = edge_segments.reshape(-1)
    pad = PPAD - P
    t_tiled = jnp.pad(t, (0, pad)).reshape(PT, NW).T.reshape(-1)
    si_tiled = jnp.pad(seg_idx, (0, pad)).reshape(PT, NW).T.reshape(-1)
    lin = jnp.linspace(0.0, 1.0, N_SAMPLES, dtype=jnp.float32)
    lin_pad = jnp.concatenate(
        [lin, jnp.full((SPAD - N_SAMPLES,), 1.0, jnp.float32)])
    outdx, outdy, outsum = _sc_call(quad, segs_flat, t_tiled, si_tiled,
                                    lin_pad, P)
    dx = outdx.reshape(NW, PT).T.reshape(-1)[:P]
    dy = outdy.reshape(NW, PT).T.reshape(-1)[:P]
    deltas = jnp.stack([dx, dy], axis=1)
    loss = jnp.sum(outsum) / np.float32(P)
    return loss, deltas


# speedup vs baseline: 1.0069x; 1.0069x over previous
import functools

import jax
import jax.numpy as jnp
import numpy as np
from jax import lax
from jax.experimental import pallas as pl
from jax.experimental.pallas import tpu as pltpu
from jax.experimental.pallas import tpu_sc as plsc

N_SAMPLES = 1000
REACH = 10.0
MAX_DIST = 0.05
IMG = 512

NC = 2
NS = 16
NW = NC * NS
PT = 128
PPAD = NW * PT
NCHUNK = 64
SPAD = NCHUNK * 16
BIGF = np.float32(3.4e38)
BIGI = np.int32(2**30)


def _vsqrt(x):
    r = lax.bitcast_convert_type(
        jnp.int32(0x5F3759DF) - (lax.bitcast_convert_type(x, jnp.int32) >> 1),
        jnp.float32)
    r = r * (1.5 - 0.5 * x * r * r)
    r = r * (1.5 - 0.5 * x * r * r)
    r = r * (1.5 - 0.5 * x * r * r)
    s = x * r
    return 0.5 * (s + x / s)


def _vsqrt_precise(x):
    s = _vsqrt(x)
    c = jnp.float32(4097.0)
    hi = (s * c) - ((s * c) - s)
    lo = s - hi
    s2 = hi * hi + (2.0 * hi * lo + lo * lo)
    resid = (x - s2)
    return s + resid / (2.0 * s)


_DNUMS = lax.GatherDimensionNumbers(
    offset_dims=(), collapsed_slice_dims=(0,), start_index_map=(0,))


def _perm(x, p):
    return lax.gather(x, p[:, None], _DNUMS, (1,),
                      indices_are_sorted=False, unique_indices=False,
                      mode=lax.GatherScatterMode.PROMISE_IN_BOUNDS)


def _lane_argmin(val, idx, i16):
    for s in (8, 4, 2, 1):
        p = i16 ^ s
        sv = _perm(val, p)
        si = _perm(idx, p)
        take = (sv < val) | ((sv == val) & (si < idx))
        val = jnp.where(take, sv, val)
        idx = jnp.where(take, si, idx)
    return val[0], idx[0]


def _sc_kernel_body(n_valid, quad_hbm, segs_hbm, t_hbm, si_hbm, lin_hbm,
                    outdx_hbm, outdy_hbm, outsum_hbm,
                    lin_v, segs_v, tt_v, si_v,
                    idx0_v, idx1_v, idx2_v, idx3_v,
                    land0_v, land1_v, land2_v, land3_v,
                    wx0_v, wx1_v, wx2_v, wx3_v,
                    wy0_v, wy1_v, wy2_v, wy3_v,
                    dd0_v, dd1_v, dd2_v, dd3_v,
                    outx_v, outy_v, sums_v, sem0, sem1, sem2, sem3):
    wid = lax.axis_index("s") * NC + lax.axis_index("c")
    base = wid * PT
    i16 = lax.iota(jnp.int32, 16)
    rep4 = i16 >> 2
    lane4 = i16 & 3
    m1 = (i16 & 1) == 1
    m2 = (i16 & 2) == 2

    pltpu.sync_copy(lin_hbm, lin_v.at[pl.ds(0, SPAD)])
    pltpu.sync_copy(segs_hbm, segs_v.at[pl.ds(0, 1024)])
    pltpu.sync_copy(t_hbm.at[pl.ds(base, PT)], tt_v.at[pl.ds(0, PT)])
    pltpu.sync_copy(si_hbm.at[pl.ds(base, PT)], si_v.at[pl.ds(0, PT)])

    for g in range(PT // 16):
        outx_v[pl.ds(g * 16, 16)] = jnp.zeros((16,), jnp.float32)
        outy_v[pl.ds(g * 16, 16)] = jnp.zeros((16,), jnp.float32)

    def prep(p):
        si = si_v[pl.ds(p, 16)][0]
        s4 = segs_v[pl.ds(si * 4, 16)]
        tt = jnp.full((16,), tt_v[pl.ds(p, 16)][0])
        rx0 = jnp.full((16,), s4[0])
        ry0 = jnp.full((16,), s4[1])
        rx1 = jnp.full((16,), s4[2])
        ry1 = jnp.full((16,), s4[3])
        ax0 = (rx0 * 0.5 + 0.5) * np.float32(IMG)
        ay0 = (0.5 - ry0 * 0.5) * np.float32(IMG)
        ax1 = (rx1 * 0.5 + 0.5) * np.float32(IMG)
        ay1 = (0.5 - ry1 * 0.5) * np.float32(IMG)
        px = (1.0 - tt) * ax0 + tt * ax1
        py = (1.0 - tt) * ay0 + tt * ay1
        dirx = ax1 - ax0
        diry = ay1 - ay0
        nx = 0.0 - diry
        ny = dirx
        nrm = _vsqrt_precise(jnp.maximum(nx * nx + ny * ny,
                                         jnp.float32(1e-37)))
        den = jnp.maximum(nrm, jnp.float32(1e-12))
        dvx = nx / den
        dvy = ny / den
        cx = jnp.abs(dvx) <= 1e-3
        cy = jnp.abs(dvy) <= 1e-3
        sdx = jnp.where(cx, jnp.float32(1.0), dvx)
        sdy = jnp.where(cy, jnp.float32(1.0), dvy)
        tl = jnp.where(cx, -BIGF, (0.0 - px) / sdx)
        tr = jnp.where(cx, BIGF, (np.float32(IMG - 1) - px) / sdx)
        tp = jnp.where(cy, -BIGF, (0.0 - py) / sdy)
        tb = jnp.where(cy, BIGF, (np.float32(IMG - 1) - py) / sdy)
        tmin = jnp.maximum(tl, tp)
        tmax = jnp.minimum(tr, tb)
        i1x = px + tmin * dvx
        i1y = py + tmin * dvy
        d12x = (px + tmax * dvx) - i1x
        d12y = (py + tmax * dvy) - i1y
        return i1x, i1y, d12x, d12y, px, py

    def a_side(p, idxS, wxS, wyS, ddS, landS, semS):
        i1x, i1y, d12x, d12y, px, py = prep(p)

        def phase_a(c, carry):
            dmin, dmini = carry
            sl = pl.ds(c * 16, 16)
            l = lin_v[sl]
            sx = l * d12x + i1x
            sy = l * d12y + i1y
            sx = jnp.minimum(jnp.maximum(sx, jnp.float32(0.0)),
                             np.float32(IMG - 1))
            sy = jnp.minimum(jnp.maximum(sy, jnp.float32(0.0)),
                             np.float32(IMG - 1))
            xi = sx.astype(jnp.int32)
            yi = sy.astype(jnp.int32)
            wxS[sl] = sx - xi.astype(jnp.float32)
            wyS[sl] = sy - yi.astype(jnp.float32)
            idxS[sl] = yi * IMG + xi
            ddx = sx - px
            ddy = sy - py
            dist = _vsqrt(jnp.maximum(ddx * ddx + ddy * ddy,
                                      jnp.float32(1e-37)))
            dist = jnp.where((ddx == 0.0) & (ddy == 0.0),
                             jnp.float32(0.0), dist)
            ddS[sl] = dist
            gi = c * 16 + i16
            dm = jnp.where(gi < N_SAMPLES, dist, BIGF)
            lt = dm < dmin
            dmin = jnp.where(lt, dm, dmin)
            dmini = jnp.where(lt, gi, dmini)
            return dmin, dmini

        dmin, dmini = lax.fori_loop(
            0, NCHUNK, phase_a,
            (jnp.full((16,), BIGF), jnp.full((16,), BIGI)), unroll=False)
        _, ref_ind = _lane_argmin(dmin, dmini, i16)
        pltpu.async_copy(quad_hbm.at[idxS], landS, semS)
        return ref_ind

    def b_side(p, ref_ind, idxS, wxS, wyS, ddS, landS, semS, sum_dd):
        pltpu.make_async_copy(quad_hbm.at[idxS], landS, semS).wait()

        def phase_b(c, carry):
            amin, amini = carry
            sl = pl.ds(c * 16, 16)
            wx = wxS[sl]
            wy = wyS[sl]
            dist = ddS[sl]
            vals16 = jnp.zeros((16,), jnp.float32)
            for b in range(4):
                s0 = c * 16 + 4 * b
                q0 = landS[s0, :]
                q1 = landS[s0 + 1, :]
                q2 = landS[s0 + 2, :]
                q3 = landS[s0 + 3, :]
                packed = jnp.where(
                    rep4 == 0, _perm(q0, lane4),
                    jnp.where(rep4 == 1, _perm(q1, lane4),
                              jnp.where(rep4 == 2, _perm(q2, lane4),
                                        _perm(q3, lane4))))
                wsel = 4 * b + rep4
                wxq = _perm(wx, wsel)
                wyq = _perm(wy, wsel)
                a = jnp.where(m1, wxq, 1.0 - wxq)
                bb = jnp.where(m2, wyq, 1.0 - wyq)
                prod = (a * bb) * packed
                t1 = prod + _perm(prod, i16 ^ 1)
                t2 = t1 + _perm(t1, i16 ^ 2)
                vals16 = jnp.where(rep4 == b,
                                   _perm(t2, 4 * ((i16 - 4 * b) & 3)),
                                   vals16)
            vals = 0.0 - vals16
            gi = c * 16 + i16
            vals = vals + jnp.where(gi == ref_ind, jnp.float32(-1e-6),
                                    jnp.float32(0.0))
            adj = vals * jnp.exp(-dist / np.float32(REACH))
            am = jnp.where(gi < N_SAMPLES, adj, BIGF)
            lt = am < amin
            amin = jnp.where(lt, am, amin)
            amini = jnp.where(lt, gi, amini)
            return amin, amini

        amin, amini = lax.fori_loop(
            0, NCHUNK, phase_b,
            (jnp.full((16,), BIGF), jnp.full((16,), BIGI)), unroll=False)
        _, m = _lane_argmin(amin, amini, i16)

        i1x, i1y, d12x, d12y, px, py = prep(p)
        lm = jnp.full((16,), lin_v[pl.ds(m, 16)][0])
        smx = jnp.minimum(jnp.maximum(lm * d12x + i1x, jnp.float32(0.0)),
                          np.float32(IMG - 1))
        smy = jnp.minimum(jnp.maximum(lm * d12y + i1y, jnp.float32(0.0)),
                          np.float32(IMG - 1))
        dex = smx - px
        dey = smy - py
        rx = dex / np.float32(IMG) * 2.0
        ry = dey / np.float32(IMG) * 2.0
        dd = _vsqrt(jnp.maximum(rx * rx + ry * ry, jnp.float32(1e-37)))
        dd = jnp.where((dex == 0.0) & (dey == 0.0), jnp.float32(0.0), dd)
        keep = dd < np.float32(MAX_DIST)
        gpid_v = jnp.full((16,), p * NW + wid, jnp.int32)
        keepv = keep & (gpid_v < n_valid)
        dex_m = jnp.where(keepv, dex, jnp.float32(0.0))
        dey_m = jnp.where(keepv, dey, jnp.float32(0.0))
        dd_m = jnp.where(keepv, dd, jnp.float32(0.0))
        row = (p >> 4) * 16
        lane = jnp.full((16,), p & 15, jnp.int32)
        ox = outx_v[pl.ds(row, 16)]
        oy = outy_v[pl.ds(row, 16)]
        outx_v[pl.ds(row, 16)] = jnp.where(i16 == lane, dex_m, ox)
        outy_v[pl.ds(row, 16)] = jnp.where(i16 == lane, dey_m, oy)
        return sum_dd + dd_m

    slots = ((idx0_v, wx0_v, wy0_v, dd0_v, land0_v, sem0),
             (idx1_v, wx1_v, wy1_v, dd1_v, land1_v, sem1),
             (idx2_v, wx2_v, wy2_v, dd2_v, land2_v, sem2),
             (idx3_v, wx3_v, wy3_v, dd3_v, land3_v, sem3))

    ri0 = a_side(jnp.int32(0), *slots[0])
    ri1 = a_side(jnp.int32(1), *slots[1])

    def pipe_body(i, carry):
        sum_dd, ri0, ri1 = carry
        p = i * 4
        ri2 = a_side(p + 2, *slots[2])
        ri3 = a_side(p + 3, *slots[3])
        sum_dd = b_side(p, ri0, *slots[0], sum_dd)
        sum_dd = b_side(p + 1, ri1, *slots[1], sum_dd)
        ri0 = a_side(p + 4, *slots[0])
        ri1 = a_side(p + 5, *slots[1])
        sum_dd = b_side(p + 2, ri2, *slots[2], sum_dd)
        sum_dd = b_side(p + 3, ri3, *slots[3], sum_dd)
        return sum_dd, ri0, ri1

    sum_dd, ri0, ri1 = lax.fori_loop(
        0, (PT - 8) // 4, pipe_body,
        (jnp.zeros((16,), jnp.float32), ri0, ri1), unroll=False)
    e = jnp.int32(PT - 8)
    ri2 = a_side(e + 2, *slots[2])
    ri3 = a_side(e + 3, *slots[3])
    sum_dd = b_side(e, ri0, *slots[0], sum_dd)
    sum_dd = b_side(e + 1, ri1, *slots[1], sum_dd)
    ri0 = a_side(e + 4, *slots[0])
    ri1 = a_side(e + 5, *slots[1])
    sum_dd = b_side(e + 2, ri2, *slots[2], sum_dd)
    sum_dd = b_side(e + 3, ri3, *slots[3], sum_dd)
    ri2 = a_side(e + 6, *slots[2])
    ri3 = a_side(e + 7, *slots[3])
    sum_dd = b_side(e + 4, ri0, *slots[0], sum_dd)
    sum_dd = b_side(e + 5, ri1, *slots[1], sum_dd)
    sum_dd = b_side(e + 6, ri2, *slots[2], sum_dd)
    sum_dd = b_side(e + 7, ri3, *slots[3], sum_dd)

    sums_v[pl.ds(0, 16)] = jnp.where(i16 == 0, sum_dd,
                                     jnp.zeros((16,), jnp.float32))
    pltpu.sync_copy(outx_v, outdx_hbm.at[pl.ds(base, PT)])
    pltpu.sync_copy(outy_v, outdy_hbm.at[pl.ds(base, PT)])
    pltpu.sync_copy(sums_v, outsum_hbm.at[pl.ds(wid * 16, 16)])


@functools.partial(jax.jit, static_argnums=(5,))
def _sc_call(quad, segs_flat, t_tiled, si_tiled, lin_pad, n_valid):
    mesh = plsc.VectorSubcoreMesh(core_axis_name="c", subcore_axis_name="s")
    f32 = jnp.float32
    kern = functools.partial(
        pl.kernel, mesh=mesh,
        compiler_params=pltpu.CompilerParams(use_tc_tiling_on_sc=False),
        out_type=[
            jax.ShapeDtypeStruct((PPAD,), f32),
            jax.ShapeDtypeStruct((PPAD,), f32),
            jax.ShapeDtypeStruct((NW * 16,), f32),
        ],
        scratch_types=(
            [pltpu.VMEM((SPAD + 16,), f32),
             pltpu.VMEM((1024 + 16,), f32),
             pltpu.VMEM((PT + 16,), f32),
             pltpu.VMEM((PT + 16,), jnp.int32)]
            + [pltpu.VMEM((SPAD,), jnp.int32)] * 4
            + [pltpu.VMEM((SPAD, 16), f32)] * 4
            + [pltpu.VMEM((SPAD,), f32)] * 12
            + [pltpu.VMEM((PT,), f32),
               pltpu.VMEM((PT,), f32),
               pltpu.VMEM((16,), f32)]
            + [pltpu.SemaphoreType.DMA] * 4
        ),
    )(functools.partial(_sc_kernel_body, jnp.int32(n_valid)))
    return kern(quad, segs_flat, t_tiled, si_tiled, lin_pad)


def kernel(edge_segments, edge_image, t, seg_idx):
    P = t.shape[0]
    imx = jnp.concatenate([edge_image[:, 1:], edge_image[:, -1:]], axis=1)
    imy = jnp.concatenate([edge_image[1:, :], edge_image[-1:, :]], axis=0)
    imxy = jnp.concatenate([imy[:, 1:], imy[:, -1:]], axis=1)
    quad = jnp.stack([edge_image.reshape(-1), imx.reshape(-1),
                      imy.reshape(-1), imxy.reshape(-1)], axis=1)
    quad = jnp.pad(quad, ((0, 0), (0, 12)))
    segs_flat = edge_segments.reshape(-1)
    pad = PPAD - P
    t_tiled = jnp.pad(t, (0, pad)).reshape(PT, NW).T.reshape(-1)
    si_tiled = jnp.pad(seg_idx, (0, pad)).reshape(PT, NW).T.reshape(-1)
    lin = jnp.linspace(0.0, 1.0, N_SAMPLES, dtype=jnp.float32)
    lin_pad = jnp.concatenate(
        [lin, jnp.full((SPAD - N_SAMPLES,), 1.0, jnp.float32)])
    outdx, outdy, outsum = _sc_call(quad, segs_flat, t_tiled, si_tiled,
                                    lin_pad, P)
    dx = outdx.reshape(NW, PT).T.reshape(-1)[:P]
    dy = outdy.reshape(NW, PT).T.reshape(-1)[:P]
    deltas = jnp.stack([dx, dy], axis=1)
    loss = jnp.sum(outsum) / np.float32(P)
    return loss, deltas
